# baseline (speedup 1.0000x reference)
import functools

import jax
import jax.numpy as jnp
from jax import lax
from jax.experimental import pallas as pl
from jax.experimental.pallas import tpu as pltpu
from jax.experimental.pallas import tpu_sc as plsc

NUM_CORES = 2
NUM_SUBCORES = 16
NUM_WORKERS = NUM_CORES * NUM_SUBCORES
G = 13
LANES = 16
CHUNK = 64


@jax.jit
def _lookup(indices, embeds):
    (B,) = indices.shape
    V, D = embeds.shape
    b_per_w = B // NUM_WORKERS
    tab_t = embeds.T

    mesh = plsc.VectorSubcoreMesh(core_axis_name="c", subcore_axis_name="s")

    part_n = b_per_w // 4

    @functools.partial(
        pl.kernel,
        mesh=mesh,
        out_type=jax.ShapeDtypeStruct((B, D), jnp.float32),
        scratch_types=[
            pltpu.VMEM((b_per_w,), jnp.int32),
            pltpu.VMEM((G, D, 128), jnp.float32),
            pltpu.VMEM((part_n, D), jnp.float32),
            pltpu.SemaphoreType.DMA((G,)),
        ],
        compiler_params=pltpu.CompilerParams(needs_layout_passes=False),
    )
    def body(idx_hbm, tab_hbm, out_hbm, idx_v, blocks_v, rows_v, sems):
        wid = lax.axis_index("s") * NUM_CORES + lax.axis_index("c")
        base = wid * b_per_w
        pltpu.sync_copy(idx_hbm.at[pl.ds(base, b_per_w)], idx_v)

        for h in range(4):
            hb = h * part_n

            def chunk(c, _):
                off = c * CHUNK
                vecs = [
                    idx_v[pl.ds(hb + off + v * LANES, LANES)]
                    for v in range(CHUNK // LANES)
                ]
                qvs = [jnp.right_shift(v, 7) for v in vecs]
                ccvs = [jnp.bitwise_and(v, 127) for v in vecs]

                def fire(i):
                    pltpu.async_copy(
                        tab_hbm.at[:, pl.ds(qvs[i // LANES][i % LANES] * 128, 128)],
                        blocks_v.at[i % G],
                        sems.at[i % G],
                    )

                def proc(i):
                    pltpu.make_async_copy(
                        tab_hbm.at[:, pl.ds(0, 128)],
                        blocks_v.at[i % G],
                        sems.at[i % G],
                    ).wait()
                    cc = jnp.full(
                        (LANES,), ccvs[i // LANES][i % LANES], dtype=jnp.int32
                    )
                    for k in range(D // LANES):
                        dv = lax.iota(jnp.int32, LANES) + (k * LANES)
                        col = plsc.load_gather(blocks_v.at[i % G], [dv, cc])
                        rows_v[off + i, pl.ds(k * LANES, LANES)] = col

                for i in range(G):
                    fire(i)
                for i in range(G, CHUNK):
                    proc(i - G)
                    fire(i)
                for i in range(CHUNK - G, CHUNK):
                    proc(i)
                return 0

            lax.fori_loop(0, part_n // CHUNK, chunk, 0)
            pltpu.sync_copy(rows_v, out_hbm.at[pl.ds(base + hb, part_n)])

    return body(indices, tab_t)


def kernel(indices, embeds):
    return _lookup(indices.astype(jnp.int32), embeds)


# speedup vs baseline: 1.0394x; 1.0394x over previous
import functools

import jax
import jax.numpy as jnp
from jax import lax
from jax.experimental import pallas as pl
from jax.experimental.pallas import tpu as pltpu
from jax.experimental.pallas import tpu_sc as plsc

NUM_CORES = 2
NUM_SUBCORES = 16
NUM_WORKERS = NUM_CORES * NUM_SUBCORES
G = 11
LANES = 16
CHUNK = 64


@jax.jit
def _lookup(indices, embeds):
    (B,) = indices.shape
    V, D = embeds.shape
    b_per_w = B // NUM_WORKERS
    tab_t = embeds.T

    mesh = plsc.VectorSubcoreMesh(core_axis_name="c", subcore_axis_name="s")

    part_n = b_per_w // 2

    @functools.partial(
        pl.kernel,
        mesh=mesh,
        out_type=jax.ShapeDtypeStruct((B, D), jnp.float32),
        scratch_types=[
            pltpu.VMEM((b_per_w,), jnp.int32),
            pltpu.VMEM((G, D, 128), jnp.float32),
            pltpu.VMEM((part_n, D), jnp.float32),
            pltpu.SemaphoreType.DMA((G,)),
        ],
        compiler_params=pltpu.CompilerParams(needs_layout_passes=False),
    )
    def body(idx_hbm, tab_hbm, out_hbm, idx_v, blocks_v, rows_v, sems):
        wid = lax.axis_index("s") * NUM_CORES + lax.axis_index("c")
        base = wid * b_per_w
        pltpu.sync_copy(idx_hbm.at[pl.ds(base, b_per_w)], idx_v)

        for h in range(2):
            hb = h * part_n

            def chunk(c, _):
                off = c * CHUNK
                vecs = [
                    idx_v[pl.ds(hb + off + v * LANES, LANES)]
                    for v in range(CHUNK // LANES)
                ]
                qvs = [jnp.right_shift(v, 7) for v in vecs]
                ccvs = [jnp.bitwise_and(v, 127) for v in vecs]

                def fire(i):
                    pltpu.async_copy(
                        tab_hbm.at[:, pl.ds(qvs[i // LANES][i % LANES] * 128, 128)],
                        blocks_v.at[i % G],
                        sems.at[i % G],
                    )

                def proc(i):
                    pltpu.make_async_copy(
                        tab_hbm.at[:, pl.ds(0, 128)],
                        blocks_v.at[i % G],
                        sems.at[i % G],
                    ).wait()
                    cc = jnp.full(
                        (LANES,), ccvs[i // LANES][i % LANES], dtype=jnp.int32
                    )
                    for k in range(D // LANES):
                        dv = lax.iota(jnp.int32, LANES) + (k * LANES)
                        col = plsc.load_gather(blocks_v.at[i % G], [dv, cc])
                        rows_v[off + i, pl.ds(k * LANES, LANES)] = col

                for i in range(G):
                    fire(i)
                for i in range(G, CHUNK):
                    proc(i - G)
                    fire(i)
                for i in range(CHUNK - G, CHUNK):
                    proc(i)
                return 0

            lax.fori_loop(0, part_n // CHUNK, chunk, 0)
            pltpu.sync_copy(rows_v, out_hbm.at[pl.ds(base + hb, part_n)])

    return body(indices, tab_t)


def kernel(indices, embeds):
    return _lookup(indices.astype(jnp.int32), embeds)
